# baseline (speedup 1.0000x reference)
import functools

import jax
import jax.numpy as jnp
from jax.experimental import pallas as pl
from jax.experimental.pallas import tpu as pltpu
from jax.experimental.pallas import tpu_sc as plsc

B = 8
N = 4096
S = 1024
K = 16
IN_DIM = 128
OUT_DIM = 256
FPAD = 144


def _fps_body(x_ref, y_ref, z_ref, idx_ref, nx_ref, ny_ref, nz_ref):
    X = x_ref[...]
    Y = y_ref[...]
    Z = z_ref[...]
    col = (jax.lax.broadcasted_iota(jnp.int32, (B, N), 1).astype(jnp.float32)
           + X * 0.0)
    col_s = (jax.lax.broadcasted_iota(jnp.int32, (B, S), 1).astype(jnp.float32)
             + X[:, :S] * 0.0)

    def body(i, state):
        dists, far, idx_acc, nx, ny, nz = state
        sel = col == far
        cx = jnp.sum(jnp.where(sel, X, 0.0), axis=1, keepdims=True)
        cy = jnp.sum(jnp.where(sel, Y, 0.0), axis=1, keepdims=True)
        cz = jnp.sum(jnp.where(sel, Z, 0.0), axis=1, keepdims=True)
        dx = X - cx
        dy = Y - cy
        dz = Z - cz
        d = (dx * dx + dy * dy) + dz * dz
        dists = jnp.minimum(dists, d)
        hot_f = (col_s == i.astype(jnp.float32)).astype(jnp.float32)
        idx_acc = idx_acc + far * hot_f
        nx = nx + cx * hot_f
        ny = ny + cy * hot_f
        nz = nz + cz * hot_f
        m = jnp.max(dists, axis=1, keepdims=True)
        far = jnp.min(jnp.where(dists == m, col, jnp.float32(N)),
                      axis=1, keepdims=True)
        return dists, far, idx_acc, nx, ny, nz

    dists0 = jnp.full((B, N), 1e10, dtype=jnp.float32)
    far0 = jnp.zeros((B, 1), dtype=jnp.float32)
    acc0 = jnp.zeros((B, S), dtype=jnp.float32)
    f0 = jnp.zeros((B, S), dtype=jnp.float32)
    _, _, idx_acc, nx, ny, nz = jax.lax.fori_loop(
        0, S, body, (dists0, far0, acc0, f0, f0, f0))
    idx_ref[...] = idx_acc.astype(jnp.int32)
    nx_ref[...] = nx
    ny_ref[...] = ny
    nz_ref[...] = nz


def _fps(x, y, z):
    return pl.pallas_call(
        _fps_body,
        out_shape=(
            jax.ShapeDtypeStruct((B, S), jnp.int32),
            jax.ShapeDtypeStruct((B, S), jnp.float32),
            jax.ShapeDtypeStruct((B, S), jnp.float32),
            jax.ShapeDtypeStruct((B, S), jnp.float32),
        ),
    )(x, y, z)


TILE_S = 256


def _knn_body(qx_ref, qy_ref, qz_ref, x_ref, y_ref, z_ref, out_ref):
    b = pl.program_id(0)
    qx = qx_ref[0]
    qy = qy_ref[0]
    qz = qz_ref[0]
    px = x_ref[0]
    py = y_ref[0]
    pz = z_ref[0]
    bf = lambda v: v.astype(jnp.bfloat16).astype(jnp.float32)
    dot = (bf(qx) * bf(px) + bf(qy) * bf(py)) + bf(qz) * bf(pz)
    qsq = (qx * qx + qy * qy) + qz * qz
    psq = (px * px + py * py) + pz * pz
    d = (qsq + psq) - 2.0 * dot
    col = (jax.lax.broadcasted_iota(jnp.int32, (TILE_S, N), 1).astype(
        jnp.float32) + d * 0.0)
    kcol = (jax.lax.broadcasted_iota(jnp.int32, (TILE_S, K), 1).astype(
        jnp.float32) + d[:, :K] * 0.0)

    def body(t, state):
        d, acc = state
        m = jnp.min(d, axis=1, keepdims=True)
        idx = jnp.min(jnp.where(d == m, col, jnp.float32(N)),
                      axis=1, keepdims=True)
        acc = acc + idx * (kcol == t.astype(jnp.float32)).astype(jnp.float32)
        d = d + (col == idx).astype(jnp.float32) * jnp.float32(1e30)
        return d, acc

    acc0 = jnp.zeros((TILE_S, K), dtype=jnp.float32)
    _, acc = jax.lax.fori_loop(0, K, body, (d, acc0))
    out_ref[0] = acc.astype(jnp.int32) + b * N


def _knn(qx, qy, qz, x, y, z):
    nsb = S // TILE_S
    return pl.pallas_call(
        _knn_body,
        grid=(B, nsb),
        in_specs=[
            pl.BlockSpec((1, TILE_S, 1), lambda b, i: (b, i, 0)),
            pl.BlockSpec((1, TILE_S, 1), lambda b, i: (b, i, 0)),
            pl.BlockSpec((1, TILE_S, 1), lambda b, i: (b, i, 0)),
            pl.BlockSpec((1, 1, N), lambda b, i: (b, 0, 0)),
            pl.BlockSpec((1, 1, N), lambda b, i: (b, 0, 0)),
            pl.BlockSpec((1, 1, N), lambda b, i: (b, 0, 0)),
        ],
        out_specs=pl.BlockSpec((1, TILE_S, K), lambda b, i: (b, i, 0)),
        out_shape=jax.ShapeDtypeStruct((B, S, K), jnp.int32),
    )(qx, qy, qz, x, y, z)


GATHER_WIN = 128
NUM_IDX = B * S * K
GPAD = 256


def _sc_gather(table, indices):
    mesh = plsc.VectorSubcoreMesh(core_axis_name="core",
                                  subcore_axis_name="subcore")

    @functools.partial(
        pl.kernel,
        out_type=jax.ShapeDtypeStruct((NUM_IDX, GPAD // 2), jnp.int32),
        mesh=mesh,
    )
    def kern(tab_hbm, idx_hbm, out_hbm):
        def body(i_vmem, o_vmem):
            pltpu.sync_copy(tab_hbm.at[i_vmem.at[0]], o_vmem)

        pltpu.emit_pipeline(
            body,
            grid=(NUM_IDX // GATHER_WIN,),
            in_specs=[pl.BlockSpec((1, GATHER_WIN), lambda i: (0, i))],
            out_specs=[pl.BlockSpec((GATHER_WIN, GPAD // 2), lambda i: (i, 0))],
            core_axis_name=("core", "subcore"),
            dimension_semantics=(pltpu.PARALLEL,),
        )(idx_hbm, out_hbm)

    return kern(table, indices)


TILE_Q = 128


def _mlp_body(r_ref, qx_ref, qy_ref, qz_ref, w1_ref, b1_ref, w2_ref, b2_ref,
              g_ref, bt_ref, out_ref):
    W1 = w1_ref[...]
    W2 = w2_ref[...]
    bf = lambda v: v.astype(jnp.bfloat16).astype(jnp.float32)
    h1 = jnp.dot(r_ref[...].reshape(TILE_Q * K, GPAD), W1,
                 preferred_element_type=jnp.float32) + b1_ref[...]
    corr = (bf(qx_ref[...]) * W1[0:1, :].astype(jnp.float32)
            + bf(qy_ref[...]) * W1[1:2, :].astype(jnp.float32)
            + bf(qz_ref[...]) * W1[2:3, :].astype(jnp.float32))
    h1 = h1.reshape(TILE_Q, K, OUT_DIM) - corr.reshape(TILE_Q, 1, OUT_DIM)
    h1 = jnp.maximum(h1, 0.0).reshape(TILE_Q * K, OUT_DIM)
    h2 = jnp.dot(h1.astype(jnp.bfloat16), W2.astype(jnp.bfloat16),
                 preferred_element_type=jnp.float32) + b2_ref[...]
    h = jnp.max(h2.reshape(TILE_Q, K, OUT_DIM), axis=1)
    mu = jnp.mean(h, axis=1, keepdims=True)
    c = h - mu
    var = jnp.mean(c * c, axis=1, keepdims=True)
    out_ref[...] = c * jax.lax.rsqrt(var + 1e-5) * g_ref[...] + bt_ref[...]


def _mlp(r3, qxc, qyc, qzc, w1p, b1, w2, b2, gamma, beta):
    nblk = (B * S) // TILE_Q
    return pl.pallas_call(
        _mlp_body,
        grid=(nblk,),
        in_specs=[
            pl.BlockSpec((TILE_Q, K, GPAD), lambda i: (i, 0, 0)),
            pl.BlockSpec((TILE_Q, 1), lambda i: (i, 0)),
            pl.BlockSpec((TILE_Q, 1), lambda i: (i, 0)),
            pl.BlockSpec((TILE_Q, 1), lambda i: (i, 0)),
            pl.BlockSpec((GPAD, OUT_DIM), lambda i: (0, 0)),
            pl.BlockSpec((1, OUT_DIM), lambda i: (0, 0)),
            pl.BlockSpec((OUT_DIM, OUT_DIM), lambda i: (0, 0)),
            pl.BlockSpec((1, OUT_DIM), lambda i: (0, 0)),
            pl.BlockSpec((1, OUT_DIM), lambda i: (0, 0)),
            pl.BlockSpec((1, OUT_DIM), lambda i: (0, 0)),
        ],
        out_specs=pl.BlockSpec((TILE_Q, OUT_DIM), lambda i: (i, 0)),
        out_shape=jax.ShapeDtypeStruct((B * S, OUT_DIM), jnp.float32),
    )(r3, qxc, qyc, qzc, w1p, b1, w2, b2, gamma, beta)


def kernel(xyz, feat, W1, b1, W2, b2, gamma, beta):
    x = xyz[:, :, 0]
    y = xyz[:, :, 1]
    z = xyz[:, :, 2]

    fps_idx, nx, ny, nz = _fps(x, y, z)
    new_xyz = jnp.stack([nx, ny, nz], axis=-1)

    knn_idx = _knn(nx[:, :, None], ny[:, :, None], nz[:, :, None],
                   x[:, None, :], y[:, None, :], z[:, None, :])

    table = jnp.concatenate(
        [xyz, feat, jnp.zeros((B, N, GPAD - 3 - IN_DIM), jnp.float32)],
        axis=-1).astype(jnp.bfloat16).reshape(B * N, GPAD // 2, 2)
    table = jax.lax.bitcast_convert_type(table, jnp.int32)
    w1p = jnp.concatenate(
        [W1, jnp.zeros((GPAD - 3 - IN_DIM, OUT_DIM), jnp.float32)],
        axis=0).astype(jnp.bfloat16)

    r = _sc_gather(table, knn_idx.reshape(1, NUM_IDX))
    r3 = jax.lax.bitcast_convert_type(r, jnp.bfloat16).reshape(
        B * S, K, GPAD)

    qxc = nx.reshape(B * S, 1)
    qyc = ny.reshape(B * S, 1)
    qzc = nz.reshape(B * S, 1)
    hn = _mlp(r3, qxc, qyc, qzc, w1p, b1[None, :], W2,
              b2[None, :], gamma[None, :], beta[None, :])
    return new_xyz, hn.reshape(B, S, OUT_DIM)


# speedup vs baseline: 1.8862x; 1.8862x over previous
import functools

import jax
import jax.numpy as jnp
from jax.experimental import pallas as pl
from jax.experimental.pallas import tpu as pltpu
from jax.experimental.pallas import tpu_sc as plsc

B = 8
N = 4096
S = 1024
K = 16
IN_DIM = 128
OUT_DIM = 256
FPAD = 144


def _fps_body(x_ref, y_ref, z_ref, idx_ref, nx_ref, ny_ref, nz_ref):
    X = x_ref[...]
    Y = y_ref[...]
    Z = z_ref[...]
    col = (jax.lax.broadcasted_iota(jnp.int32, (B, N), 1).astype(jnp.float32)
           + X * 0.0)
    col_s = (jax.lax.broadcasted_iota(jnp.int32, (B, S), 1).astype(jnp.float32)
             + X[:, :S] * 0.0)

    def body(i, state):
        dists, far, idx_acc, nx, ny, nz = state
        sel = col == far
        cx = jnp.sum(jnp.where(sel, X, 0.0), axis=1, keepdims=True)
        cy = jnp.sum(jnp.where(sel, Y, 0.0), axis=1, keepdims=True)
        cz = jnp.sum(jnp.where(sel, Z, 0.0), axis=1, keepdims=True)
        dx = X - cx
        dy = Y - cy
        dz = Z - cz
        d = (dx * dx + dy * dy) + dz * dz
        dists = jnp.minimum(dists, d)
        hot_f = (col_s == i.astype(jnp.float32)).astype(jnp.float32)
        idx_acc = idx_acc + far * hot_f
        nx = nx + cx * hot_f
        ny = ny + cy * hot_f
        nz = nz + cz * hot_f
        m = jnp.max(dists, axis=1, keepdims=True)
        far = jnp.min(jnp.where(dists == m, col, jnp.float32(N)),
                      axis=1, keepdims=True)
        return dists, far, idx_acc, nx, ny, nz

    dists0 = jnp.full((B, N), 1e10, dtype=jnp.float32)
    far0 = jnp.zeros((B, 1), dtype=jnp.float32)
    acc0 = jnp.zeros((B, S), dtype=jnp.float32)
    f0 = jnp.zeros((B, S), dtype=jnp.float32)
    _, _, idx_acc, nx, ny, nz = jax.lax.fori_loop(
        0, S, body, (dists0, far0, acc0, f0, f0, f0))
    idx_ref[...] = idx_acc.astype(jnp.int32)
    nx_ref[...] = nx
    ny_ref[...] = ny
    nz_ref[...] = nz


def _fps(x, y, z):
    return pl.pallas_call(
        _fps_body,
        out_shape=(
            jax.ShapeDtypeStruct((B, S), jnp.int32),
            jax.ShapeDtypeStruct((B, S), jnp.float32),
            jax.ShapeDtypeStruct((B, S), jnp.float32),
            jax.ShapeDtypeStruct((B, S), jnp.float32),
        ),
    )(x, y, z)


TILE_S = 256


def _knn_body(qx_ref, qy_ref, qz_ref, x_ref, y_ref, z_ref, out_ref):
    b = pl.program_id(0)
    qx = qx_ref[0]
    qy = qy_ref[0]
    qz = qz_ref[0]
    px = x_ref[0]
    py = y_ref[0]
    pz = z_ref[0]
    bf = lambda v: v.astype(jnp.bfloat16).astype(jnp.float32)
    dot = (bf(qx) * bf(px) + bf(qy) * bf(py)) + bf(qz) * bf(pz)
    qsq = (qx * qx + qy * qy) + qz * qz
    psq = (px * px + py * py) + pz * pz
    d = (qsq + psq) - 2.0 * dot
    col = (jax.lax.broadcasted_iota(jnp.int32, (TILE_S, N), 1).astype(
        jnp.float32) + d * 0.0)
    kcol = (jax.lax.broadcasted_iota(jnp.int32, (TILE_S, K), 1).astype(
        jnp.float32) + d[:, :K] * 0.0)

    def body(t, state):
        d, acc = state
        m = jnp.min(d, axis=1, keepdims=True)
        idx = jnp.min(jnp.where(d == m, col, jnp.float32(N)),
                      axis=1, keepdims=True)
        acc = acc + idx * (kcol == t.astype(jnp.float32)).astype(jnp.float32)
        d = d + (col == idx).astype(jnp.float32) * jnp.float32(1e30)
        return d, acc

    acc0 = jnp.zeros((TILE_S, K), dtype=jnp.float32)
    _, acc = jax.lax.fori_loop(0, K, body, (d, acc0))
    out_ref[0] = acc.astype(jnp.int32) + b * N


def _knn(qx, qy, qz, x, y, z):
    nsb = S // TILE_S
    return pl.pallas_call(
        _knn_body,
        grid=(B, nsb),
        in_specs=[
            pl.BlockSpec((1, TILE_S, 1), lambda b, i: (b, i, 0)),
            pl.BlockSpec((1, TILE_S, 1), lambda b, i: (b, i, 0)),
            pl.BlockSpec((1, TILE_S, 1), lambda b, i: (b, i, 0)),
            pl.BlockSpec((1, 1, N), lambda b, i: (b, 0, 0)),
            pl.BlockSpec((1, 1, N), lambda b, i: (b, 0, 0)),
            pl.BlockSpec((1, 1, N), lambda b, i: (b, 0, 0)),
        ],
        out_specs=pl.BlockSpec((1, TILE_S, K), lambda b, i: (b, i, 0)),
        out_shape=jax.ShapeDtypeStruct((B, S, K), jnp.int32),
    )(qx, qy, qz, x, y, z)


GATHER_WIN = 128
NUM_IDX = B * S * K
GPAD = 256


def _sc_gather(table, indices):
    mesh = plsc.VectorSubcoreMesh(core_axis_name="core",
                                  subcore_axis_name="subcore")

    @functools.partial(
        pl.kernel,
        out_type=jax.ShapeDtypeStruct((NUM_IDX, GPAD // 2), jnp.int32),
        mesh=mesh,
    )
    def kern(tab_hbm, idx_hbm, out_hbm):
        def body(i_vmem, o_vmem):
            pltpu.sync_copy(tab_hbm.at[i_vmem.at[0]], o_vmem)

        pltpu.emit_pipeline(
            body,
            grid=(NUM_IDX // GATHER_WIN,),
            in_specs=[pl.BlockSpec((1, GATHER_WIN), lambda i: (0, i))],
            out_specs=[pl.BlockSpec((GATHER_WIN, GPAD // 2), lambda i: (i, 0))],
            core_axis_name=("core", "subcore"),
            dimension_semantics=(pltpu.PARALLEL,),
        )(idx_hbm, out_hbm)

    return kern(table, indices)


TILE_Q = 128


def _mlp_body(r_ref, qx_ref, qy_ref, qz_ref, w1_ref, b1_ref, w2_ref, b2_ref,
              g_ref, bt_ref, out_ref):
    W1 = w1_ref[...]
    W2 = w2_ref[...]
    bf = lambda v: v.astype(jnp.bfloat16).astype(jnp.float32)
    ri = r_ref[...].reshape(TILE_Q * K, GPAD // 2)
    lo = jax.lax.bitcast_convert_type(ri << 16, jnp.float32)
    hi = jax.lax.bitcast_convert_type(
        ri & jnp.int32(-65536), jnp.float32)
    h1 = (jnp.dot(lo.astype(jnp.bfloat16), W1[:GPAD // 2, :],
                  preferred_element_type=jnp.float32)
          + jnp.dot(hi.astype(jnp.bfloat16), W1[GPAD // 2:, :],
                    preferred_element_type=jnp.float32)) + b1_ref[...]
    corr = (bf(qx_ref[...]) * W1[0:1, :].astype(jnp.float32)
            + bf(qy_ref[...]) * W1[1:2, :].astype(jnp.float32)
            + bf(qz_ref[...]) * W1[2:3, :].astype(jnp.float32))
    h1 = h1.reshape(TILE_Q, K, OUT_DIM) - corr.reshape(TILE_Q, 1, OUT_DIM)
    h1 = jnp.maximum(h1, 0.0).reshape(TILE_Q * K, OUT_DIM)
    h2 = jnp.dot(h1.astype(jnp.bfloat16), W2.astype(jnp.bfloat16),
                 preferred_element_type=jnp.float32) + b2_ref[...]
    h = jnp.max(h2.reshape(TILE_Q, K, OUT_DIM), axis=1)
    mu = jnp.mean(h, axis=1, keepdims=True)
    c = h - mu
    var = jnp.mean(c * c, axis=1, keepdims=True)
    out_ref[...] = c * jax.lax.rsqrt(var + 1e-5) * g_ref[...] + bt_ref[...]


def _mlp(r3, qxc, qyc, qzc, w1p, b1, w2, b2, gamma, beta):
    nblk = (B * S) // TILE_Q
    return pl.pallas_call(
        _mlp_body,
        grid=(nblk,),
        in_specs=[
            pl.BlockSpec((TILE_Q, K, GPAD // 2), lambda i: (i, 0, 0)),
            pl.BlockSpec((TILE_Q, 1), lambda i: (i, 0)),
            pl.BlockSpec((TILE_Q, 1), lambda i: (i, 0)),
            pl.BlockSpec((TILE_Q, 1), lambda i: (i, 0)),
            pl.BlockSpec((GPAD, OUT_DIM), lambda i: (0, 0)),
            pl.BlockSpec((1, OUT_DIM), lambda i: (0, 0)),
            pl.BlockSpec((OUT_DIM, OUT_DIM), lambda i: (0, 0)),
            pl.BlockSpec((1, OUT_DIM), lambda i: (0, 0)),
            pl.BlockSpec((1, OUT_DIM), lambda i: (0, 0)),
            pl.BlockSpec((1, OUT_DIM), lambda i: (0, 0)),
        ],
        out_specs=pl.BlockSpec((TILE_Q, OUT_DIM), lambda i: (i, 0)),
        out_shape=jax.ShapeDtypeStruct((B * S, OUT_DIM), jnp.float32),
    )(r3, qxc, qyc, qzc, w1p, b1, w2, b2, gamma, beta)


def kernel(xyz, feat, W1, b1, W2, b2, gamma, beta):
    x = xyz[:, :, 0]
    y = xyz[:, :, 1]
    z = xyz[:, :, 2]

    fps_idx, nx, ny, nz = _fps(x, y, z)
    new_xyz = jnp.stack([nx, ny, nz], axis=-1)

    knn_idx = _knn(nx[:, :, None], ny[:, :, None], nz[:, :, None],
                   x[:, None, :], y[:, None, :], z[:, None, :])

    tab_bf = jnp.concatenate(
        [xyz, feat, jnp.zeros((B, N, GPAD - 3 - IN_DIM), jnp.float32)],
        axis=-1).astype(jnp.bfloat16).reshape(B * N, GPAD)
    bits = jax.lax.bitcast_convert_type(tab_bf, jnp.uint16).astype(jnp.uint32)
    table = ((bits[:, GPAD // 2:] << 16) | bits[:, :GPAD // 2]).astype(
        jnp.int32)
    w1p = jnp.concatenate(
        [W1, jnp.zeros((GPAD - 3 - IN_DIM, OUT_DIM), jnp.float32)],
        axis=0).astype(jnp.bfloat16)

    r = _sc_gather(table, knn_idx.reshape(1, NUM_IDX))
    r3 = r.reshape(B * S, K, GPAD // 2)

    qxc = nx.reshape(B * S, 1)
    qyc = ny.reshape(B * S, 1)
    qzc = nz.reshape(B * S, 1)
    hn = _mlp(r3, qxc, qyc, qzc, w1p, b1[None, :], W2,
              b2[None, :], gamma[None, :], beta[None, :])
    return new_xyz, hn.reshape(B, S, OUT_DIM)


# speedup vs baseline: 1.8952x; 1.0048x over previous
import functools

import jax
import jax.numpy as jnp
from jax.experimental import pallas as pl
from jax.experimental.pallas import tpu as pltpu
from jax.experimental.pallas import tpu_sc as plsc

B = 8
N = 4096
S = 1024
K = 16
IN_DIM = 128
OUT_DIM = 256
FPAD = 144


def _fps_body(x_ref, y_ref, z_ref, idx_ref, nx_ref, ny_ref, nz_ref):
    X = x_ref[...]
    Y = y_ref[...]
    Z = z_ref[...]
    col = (jax.lax.broadcasted_iota(jnp.int32, (B, N), 1).astype(jnp.float32)
           + X * 0.0)
    col_s = (jax.lax.broadcasted_iota(jnp.int32, (B, S), 1).astype(jnp.float32)
             + X[:, :S] * 0.0)

    def body(i, state):
        dists, far, idx_acc, nx, ny, nz = state
        sel = col == far
        cx = jnp.sum(jnp.where(sel, X, 0.0), axis=1, keepdims=True)
        cy = jnp.sum(jnp.where(sel, Y, 0.0), axis=1, keepdims=True)
        cz = jnp.sum(jnp.where(sel, Z, 0.0), axis=1, keepdims=True)
        dx = X - cx
        dy = Y - cy
        dz = Z - cz
        d = (dx * dx + dy * dy) + dz * dz
        dists = jnp.minimum(dists, d)
        hot_f = (col_s == i.astype(jnp.float32)).astype(jnp.float32)
        idx_acc = idx_acc + far * hot_f
        nx = nx + cx * hot_f
        ny = ny + cy * hot_f
        nz = nz + cz * hot_f
        m = jnp.max(dists, axis=1, keepdims=True)
        far = jnp.min(jnp.where(dists == m, col, jnp.float32(N)),
                      axis=1, keepdims=True)
        return dists, far, idx_acc, nx, ny, nz

    dists0 = jnp.full((B, N), 1e10, dtype=jnp.float32)
    far0 = jnp.zeros((B, 1), dtype=jnp.float32)
    acc0 = jnp.zeros((B, S), dtype=jnp.float32)
    f0 = jnp.zeros((B, S), dtype=jnp.float32)
    _, _, idx_acc, nx, ny, nz = jax.lax.fori_loop(
        0, S, body, (dists0, far0, acc0, f0, f0, f0))
    idx_ref[...] = idx_acc.astype(jnp.int32)
    nx_ref[...] = nx
    ny_ref[...] = ny
    nz_ref[...] = nz


def _fps(x, y, z):
    return pl.pallas_call(
        _fps_body,
        out_shape=(
            jax.ShapeDtypeStruct((B, S), jnp.int32),
            jax.ShapeDtypeStruct((B, S), jnp.float32),
            jax.ShapeDtypeStruct((B, S), jnp.float32),
            jax.ShapeDtypeStruct((B, S), jnp.float32),
        ),
    )(x, y, z)


TILE_S = 256


def _knn_body(qx_ref, qy_ref, qz_ref, x_ref, y_ref, z_ref, out_ref):
    b = pl.program_id(0)
    qx = qx_ref[0]
    qy = qy_ref[0]
    qz = qz_ref[0]
    px = x_ref[0]
    py = y_ref[0]
    pz = z_ref[0]
    bf = lambda v: v.astype(jnp.bfloat16).astype(jnp.float32)
    dot = (bf(qx) * bf(px) + bf(qy) * bf(py)) + bf(qz) * bf(pz)
    qsq = (qx * qx + qy * qy) + qz * qz
    psq = (px * px + py * py) + pz * pz
    d = (qsq + psq) - 2.0 * dot
    col = (jax.lax.broadcasted_iota(jnp.int32, (TILE_S, N), 1).astype(
        jnp.float32) + d * 0.0)
    kcol = (jax.lax.broadcasted_iota(jnp.int32, (TILE_S, K), 1).astype(
        jnp.float32) + d[:, :K] * 0.0)

    def body(t, state):
        d, acc = state
        m = jnp.min(d, axis=1, keepdims=True)
        idx = jnp.min(jnp.where(d == m, col, jnp.float32(N)),
                      axis=1, keepdims=True)
        acc = acc + idx * (kcol == t.astype(jnp.float32)).astype(jnp.float32)
        d = d + (col == idx).astype(jnp.float32) * jnp.float32(1e30)
        return d, acc

    acc0 = jnp.zeros((TILE_S, K), dtype=jnp.float32)
    _, acc = jax.lax.fori_loop(0, K, body, (d, acc0))
    out_ref[0] = acc.astype(jnp.int32) + b * N


def _knn(qx, qy, qz, x, y, z):
    nsb = S // TILE_S
    return pl.pallas_call(
        _knn_body,
        grid=(B, nsb),
        in_specs=[
            pl.BlockSpec((1, TILE_S, 1), lambda b, i: (b, i, 0)),
            pl.BlockSpec((1, TILE_S, 1), lambda b, i: (b, i, 0)),
            pl.BlockSpec((1, TILE_S, 1), lambda b, i: (b, i, 0)),
            pl.BlockSpec((1, 1, N), lambda b, i: (b, 0, 0)),
            pl.BlockSpec((1, 1, N), lambda b, i: (b, 0, 0)),
            pl.BlockSpec((1, 1, N), lambda b, i: (b, 0, 0)),
        ],
        out_specs=pl.BlockSpec((1, TILE_S, K), lambda b, i: (b, i, 0)),
        out_shape=jax.ShapeDtypeStruct((B, S, K), jnp.int32),
    )(qx, qy, qz, x, y, z)


GATHER_WIN = 256
NUM_IDX = B * S * K
GPAD = 256


def _sc_gather(table, indices):
    mesh = plsc.VectorSubcoreMesh(core_axis_name="core",
                                  subcore_axis_name="subcore")

    @functools.partial(
        pl.kernel,
        out_type=jax.ShapeDtypeStruct((NUM_IDX, GPAD // 2), jnp.int32),
        mesh=mesh,
    )
    def kern(tab_hbm, idx_hbm, out_hbm):
        def body(i_vmem, o_vmem):
            pltpu.sync_copy(tab_hbm.at[i_vmem.at[0]], o_vmem)

        pltpu.emit_pipeline(
            body,
            grid=(NUM_IDX // GATHER_WIN,),
            in_specs=[pl.BlockSpec((1, GATHER_WIN), lambda i: (0, i))],
            out_specs=[pl.BlockSpec((GATHER_WIN, GPAD // 2), lambda i: (i, 0))],
            core_axis_name=("core", "subcore"),
            dimension_semantics=(pltpu.PARALLEL,),
        )(idx_hbm, out_hbm)

    return kern(table, indices)


TILE_Q = 128


def _mlp_body(r_ref, qx_ref, qy_ref, qz_ref, w1_ref, b1_ref, w2_ref, b2_ref,
              g_ref, bt_ref, out_ref):
    W1 = w1_ref[...]
    W2 = w2_ref[...]
    bf = lambda v: v.astype(jnp.bfloat16).astype(jnp.float32)
    ri = r_ref[...].reshape(TILE_Q * K, GPAD // 2)
    lo = jax.lax.bitcast_convert_type(ri << 16, jnp.float32)
    hi = jax.lax.bitcast_convert_type(
        ri & jnp.int32(-65536), jnp.float32)
    h1 = (jnp.dot(lo.astype(jnp.bfloat16), W1[:GPAD // 2, :],
                  preferred_element_type=jnp.float32)
          + jnp.dot(hi.astype(jnp.bfloat16), W1[GPAD // 2:, :],
                    preferred_element_type=jnp.float32)) + b1_ref[...]
    corr = (bf(qx_ref[...]) * W1[0:1, :].astype(jnp.float32)
            + bf(qy_ref[...]) * W1[1:2, :].astype(jnp.float32)
            + bf(qz_ref[...]) * W1[2:3, :].astype(jnp.float32))
    h1 = h1.reshape(TILE_Q, K, OUT_DIM) - corr.reshape(TILE_Q, 1, OUT_DIM)
    h1 = jnp.maximum(h1, 0.0).reshape(TILE_Q * K, OUT_DIM)
    h2 = jnp.dot(h1.astype(jnp.bfloat16), W2.astype(jnp.bfloat16),
                 preferred_element_type=jnp.float32) + b2_ref[...]
    h = jnp.max(h2.reshape(TILE_Q, K, OUT_DIM), axis=1)
    mu = jnp.mean(h, axis=1, keepdims=True)
    c = h - mu
    var = jnp.mean(c * c, axis=1, keepdims=True)
    out_ref[...] = c * jax.lax.rsqrt(var + 1e-5) * g_ref[...] + bt_ref[...]


def _mlp(r3, qxc, qyc, qzc, w1p, b1, w2, b2, gamma, beta):
    nblk = (B * S) // TILE_Q
    return pl.pallas_call(
        _mlp_body,
        grid=(nblk,),
        in_specs=[
            pl.BlockSpec((TILE_Q, K, GPAD // 2), lambda i: (i, 0, 0)),
            pl.BlockSpec((TILE_Q, 1), lambda i: (i, 0)),
            pl.BlockSpec((TILE_Q, 1), lambda i: (i, 0)),
            pl.BlockSpec((TILE_Q, 1), lambda i: (i, 0)),
            pl.BlockSpec((GPAD, OUT_DIM), lambda i: (0, 0)),
            pl.BlockSpec((1, OUT_DIM), lambda i: (0, 0)),
            pl.BlockSpec((OUT_DIM, OUT_DIM), lambda i: (0, 0)),
            pl.BlockSpec((1, OUT_DIM), lambda i: (0, 0)),
            pl.BlockSpec((1, OUT_DIM), lambda i: (0, 0)),
            pl.BlockSpec((1, OUT_DIM), lambda i: (0, 0)),
        ],
        out_specs=pl.BlockSpec((TILE_Q, OUT_DIM), lambda i: (i, 0)),
        out_shape=jax.ShapeDtypeStruct((B * S, OUT_DIM), jnp.float32),
    )(r3, qxc, qyc, qzc, w1p, b1, w2, b2, gamma, beta)


def kernel(xyz, feat, W1, b1, W2, b2, gamma, beta):
    x = xyz[:, :, 0]
    y = xyz[:, :, 1]
    z = xyz[:, :, 2]

    fps_idx, nx, ny, nz = _fps(x, y, z)
    new_xyz = jnp.stack([nx, ny, nz], axis=-1)

    knn_idx = _knn(nx[:, :, None], ny[:, :, None], nz[:, :, None],
                   x[:, None, :], y[:, None, :], z[:, None, :])

    tab_bf = jnp.concatenate(
        [xyz, feat, jnp.zeros((B, N, GPAD - 3 - IN_DIM), jnp.float32)],
        axis=-1).astype(jnp.bfloat16).reshape(B * N, GPAD)
    bits = jax.lax.bitcast_convert_type(tab_bf, jnp.uint16).astype(jnp.uint32)
    table = ((bits[:, GPAD // 2:] << 16) | bits[:, :GPAD // 2]).astype(
        jnp.int32)
    w1p = jnp.concatenate(
        [W1, jnp.zeros((GPAD - 3 - IN_DIM, OUT_DIM), jnp.float32)],
        axis=0).astype(jnp.bfloat16)

    r = _sc_gather(table, knn_idx.reshape(1, NUM_IDX))
    r3 = r.reshape(B * S, K, GPAD // 2)

    qxc = nx.reshape(B * S, 1)
    qyc = ny.reshape(B * S, 1)
    qzc = nz.reshape(B * S, 1)
    hn = _mlp(r3, qxc, qyc, qzc, w1p, b1[None, :], W2,
              b2[None, :], gamma[None, :], beta[None, :])
    return new_xyz, hn.reshape(B, S, OUT_DIM)


# speedup vs baseline: 1.9052x; 1.0053x over previous
import functools

import jax
import jax.numpy as jnp
from jax.experimental import pallas as pl
from jax.experimental.pallas import tpu as pltpu
from jax.experimental.pallas import tpu_sc as plsc

B = 8
N = 4096
S = 1024
K = 16
IN_DIM = 128
OUT_DIM = 256
FPAD = 144


def _fps_body(x_ref, y_ref, z_ref, idx_ref, nx_ref, ny_ref, nz_ref):
    X = x_ref[...]
    Y = y_ref[...]
    Z = z_ref[...]
    col = (jax.lax.broadcasted_iota(jnp.int32, (B, N), 1).astype(jnp.float32)
           + X * 0.0)
    col_s = (jax.lax.broadcasted_iota(jnp.int32, (B, S), 1).astype(jnp.float32)
             + X[:, :S] * 0.0)

    def body(i, state):
        dists, far, idx_acc, nx, ny, nz = state
        sel = col == far
        cx = jnp.sum(jnp.where(sel, X, 0.0), axis=1, keepdims=True)
        cy = jnp.sum(jnp.where(sel, Y, 0.0), axis=1, keepdims=True)
        cz = jnp.sum(jnp.where(sel, Z, 0.0), axis=1, keepdims=True)
        dx = X - cx
        dy = Y - cy
        dz = Z - cz
        d = (dx * dx + dy * dy) + dz * dz
        dists = jnp.minimum(dists, d)
        hot_f = (col_s == i.astype(jnp.float32)).astype(jnp.float32)
        idx_acc = idx_acc + far * hot_f
        nx = nx + cx * hot_f
        ny = ny + cy * hot_f
        nz = nz + cz * hot_f
        m = jnp.max(dists, axis=1, keepdims=True)
        far = jnp.min(jnp.where(dists == m, col, jnp.float32(N)),
                      axis=1, keepdims=True)
        return dists, far, idx_acc, nx, ny, nz

    dists0 = jnp.full((B, N), 1e10, dtype=jnp.float32)
    far0 = jnp.zeros((B, 1), dtype=jnp.float32)
    acc0 = jnp.zeros((B, S), dtype=jnp.float32)
    f0 = jnp.zeros((B, S), dtype=jnp.float32)
    _, _, idx_acc, nx, ny, nz = jax.lax.fori_loop(
        0, S, body, (dists0, far0, acc0, f0, f0, f0))
    idx_ref[...] = idx_acc.astype(jnp.int32)
    nx_ref[...] = nx
    ny_ref[...] = ny
    nz_ref[...] = nz


def _fps(x, y, z):
    return pl.pallas_call(
        _fps_body,
        out_shape=(
            jax.ShapeDtypeStruct((B, S), jnp.int32),
            jax.ShapeDtypeStruct((B, S), jnp.float32),
            jax.ShapeDtypeStruct((B, S), jnp.float32),
            jax.ShapeDtypeStruct((B, S), jnp.float32),
        ),
    )(x, y, z)


TILE_S = 256


def _knn_body(b0, qx_ref, qy_ref, qz_ref, x_ref, y_ref, z_ref, out_ref):
    b = pl.program_id(0) + b0
    qx = qx_ref[0]
    qy = qy_ref[0]
    qz = qz_ref[0]
    px = x_ref[0]
    py = y_ref[0]
    pz = z_ref[0]
    bf = lambda v: v.astype(jnp.bfloat16).astype(jnp.float32)
    dot = (bf(qx) * bf(px) + bf(qy) * bf(py)) + bf(qz) * bf(pz)
    qsq = (qx * qx + qy * qy) + qz * qz
    psq = (px * px + py * py) + pz * pz
    d = (qsq + psq) - 2.0 * dot
    col = (jax.lax.broadcasted_iota(jnp.int32, (TILE_S, N), 1).astype(
        jnp.float32) + d * 0.0)
    kcol = (jax.lax.broadcasted_iota(jnp.int32, (TILE_S, K), 1).astype(
        jnp.float32) + d[:, :K] * 0.0)

    def body(t, state):
        d, acc = state
        m = jnp.min(d, axis=1, keepdims=True)
        idx = jnp.min(jnp.where(d == m, col, jnp.float32(N)),
                      axis=1, keepdims=True)
        acc = acc + idx * (kcol == t.astype(jnp.float32)).astype(jnp.float32)
        d = d + (col == idx).astype(jnp.float32) * jnp.float32(1e30)
        return d, acc

    acc0 = jnp.zeros((TILE_S, K), dtype=jnp.float32)
    _, acc = jax.lax.fori_loop(0, K, body, (d, acc0))
    out_ref[0] = acc.astype(jnp.int32) + b * N


def _knn(qx, qy, qz, x, y, z, b0, nb):
    nsb = S // TILE_S
    qmap = lambda b, i: (b + b0, i, 0)
    pmap = lambda b, i: (b + b0, 0, 0)
    return pl.pallas_call(
        functools.partial(_knn_body, b0),
        grid=(nb, nsb),
        in_specs=[
            pl.BlockSpec((1, TILE_S, 1), qmap),
            pl.BlockSpec((1, TILE_S, 1), qmap),
            pl.BlockSpec((1, TILE_S, 1), qmap),
            pl.BlockSpec((1, 1, N), pmap),
            pl.BlockSpec((1, 1, N), pmap),
            pl.BlockSpec((1, 1, N), pmap),
        ],
        out_specs=pl.BlockSpec((1, TILE_S, K), lambda b, i: (b, i, 0)),
        out_shape=jax.ShapeDtypeStruct((nb, S, K), jnp.int32),
    )(qx, qy, qz, x, y, z)


GATHER_WIN = 256
NUM_IDX = B * S * K
GPAD = 256


def _sc_gather(table, indices, m):
    mesh = plsc.VectorSubcoreMesh(core_axis_name="core",
                                  subcore_axis_name="subcore")

    @functools.partial(
        pl.kernel,
        out_type=jax.ShapeDtypeStruct((m, GPAD // 2), jnp.int32),
        mesh=mesh,
    )
    def kern(tab_hbm, idx_hbm, out_hbm):
        def body(i_vmem, o_vmem):
            pltpu.sync_copy(tab_hbm.at[i_vmem.at[0]], o_vmem)

        pltpu.emit_pipeline(
            body,
            grid=(m // GATHER_WIN,),
            in_specs=[pl.BlockSpec((1, GATHER_WIN), lambda i: (0, i))],
            out_specs=[pl.BlockSpec((GATHER_WIN, GPAD // 2), lambda i: (i, 0))],
            core_axis_name=("core", "subcore"),
            dimension_semantics=(pltpu.PARALLEL,),
        )(idx_hbm, out_hbm)

    return kern(table, indices)


TILE_Q = 128


def _mlp_body(r_ref, qx_ref, qy_ref, qz_ref, w1_ref, b1_ref, w2_ref, b2_ref,
              g_ref, bt_ref, out_ref):
    W1 = w1_ref[...]
    W2 = w2_ref[...]
    bf = lambda v: v.astype(jnp.bfloat16).astype(jnp.float32)
    ri = r_ref[...].reshape(TILE_Q * K, GPAD // 2)
    lo = jax.lax.bitcast_convert_type(ri << 16, jnp.float32)
    hi = jax.lax.bitcast_convert_type(
        ri & jnp.int32(-65536), jnp.float32)
    h1 = (jnp.dot(lo.astype(jnp.bfloat16), W1[:GPAD // 2, :],
                  preferred_element_type=jnp.float32)
          + jnp.dot(hi.astype(jnp.bfloat16), W1[GPAD // 2:, :],
                    preferred_element_type=jnp.float32)) + b1_ref[...]
    corr = (bf(qx_ref[...]) * W1[0:1, :].astype(jnp.float32)
            + bf(qy_ref[...]) * W1[1:2, :].astype(jnp.float32)
            + bf(qz_ref[...]) * W1[2:3, :].astype(jnp.float32))
    h1 = h1.reshape(TILE_Q, K, OUT_DIM) - corr.reshape(TILE_Q, 1, OUT_DIM)
    h1 = jnp.maximum(h1, 0.0).reshape(TILE_Q * K, OUT_DIM)
    h2 = jnp.dot(h1.astype(jnp.bfloat16), W2.astype(jnp.bfloat16),
                 preferred_element_type=jnp.float32) + b2_ref[...]
    h = jnp.max(h2.reshape(TILE_Q, K, OUT_DIM), axis=1)
    mu = jnp.mean(h, axis=1, keepdims=True)
    c = h - mu
    var = jnp.mean(c * c, axis=1, keepdims=True)
    out_ref[...] = c * jax.lax.rsqrt(var + 1e-5) * g_ref[...] + bt_ref[...]


def _mlp(r3, qxc, qyc, qzc, w1p, b1, w2, b2, gamma, beta):
    nq = r3.shape[0]
    nblk = nq // TILE_Q
    return pl.pallas_call(
        _mlp_body,
        grid=(nblk,),
        in_specs=[
            pl.BlockSpec((TILE_Q, K, GPAD // 2), lambda i: (i, 0, 0)),
            pl.BlockSpec((TILE_Q, 1), lambda i: (i, 0)),
            pl.BlockSpec((TILE_Q, 1), lambda i: (i, 0)),
            pl.BlockSpec((TILE_Q, 1), lambda i: (i, 0)),
            pl.BlockSpec((GPAD, OUT_DIM), lambda i: (0, 0)),
            pl.BlockSpec((1, OUT_DIM), lambda i: (0, 0)),
            pl.BlockSpec((OUT_DIM, OUT_DIM), lambda i: (0, 0)),
            pl.BlockSpec((1, OUT_DIM), lambda i: (0, 0)),
            pl.BlockSpec((1, OUT_DIM), lambda i: (0, 0)),
            pl.BlockSpec((1, OUT_DIM), lambda i: (0, 0)),
        ],
        out_specs=pl.BlockSpec((TILE_Q, OUT_DIM), lambda i: (i, 0)),
        out_shape=jax.ShapeDtypeStruct((nq, OUT_DIM), jnp.float32),
    )(r3, qxc, qyc, qzc, w1p, b1, w2, b2, gamma, beta)


def kernel(xyz, feat, W1, b1, W2, b2, gamma, beta):
    x = xyz[:, :, 0]
    y = xyz[:, :, 1]
    z = xyz[:, :, 2]

    fps_idx, nx, ny, nz = _fps(x, y, z)
    new_xyz = jnp.stack([nx, ny, nz], axis=-1)

    tab_bf = jnp.concatenate(
        [xyz, feat, jnp.zeros((B, N, GPAD - 3 - IN_DIM), jnp.float32)],
        axis=-1).astype(jnp.bfloat16).reshape(B * N, GPAD)
    bits = jax.lax.bitcast_convert_type(tab_bf, jnp.uint16).astype(jnp.uint32)
    table = ((bits[:, GPAD // 2:] << 16) | bits[:, :GPAD // 2]).astype(
        jnp.int32)
    w1p = jnp.concatenate(
        [W1, jnp.zeros((GPAD - 3 - IN_DIM, OUT_DIM), jnp.float32)],
        axis=0).astype(jnp.bfloat16)

    qx3 = nx[:, :, None]
    qy3 = ny[:, :, None]
    qz3 = nz[:, :, None]
    px3 = x[:, None, :]
    py3 = y[:, None, :]
    pz3 = z[:, None, :]

    NB = B // 2
    halves = []
    for h in range(2):
        b0 = h * NB
        knn_idx = _knn(qx3, qy3, qz3, px3, py3, pz3, b0, NB)
        m = NB * S * K
        r = _sc_gather(table, knn_idx.reshape(1, m), m)
        r3 = r.reshape(NB * S, K, GPAD // 2)
        qxc = nx[b0:b0 + NB].reshape(NB * S, 1)
        qyc = ny[b0:b0 + NB].reshape(NB * S, 1)
        qzc = nz[b0:b0 + NB].reshape(NB * S, 1)
        halves.append(_mlp(r3, qxc, qyc, qzc, w1p, b1[None, :], W2,
                           b2[None, :], gamma[None, :], beta[None, :]))
    hn = jnp.concatenate(halves, axis=0)
    return new_xyz, hn.reshape(B, S, OUT_DIM)
